# baseline (speedup 1.0000x reference)
import jax
import jax.numpy as jnp
from jax.experimental import pallas as pl
from jax.experimental.pallas import tpu as pltpu

_B = 1024
_BB = 8
_H = 8
_K = 256
_E = 200
_V = 256
_TOPK = 32
_ROWS = _BB * _H


def _dnd_body(state_ref, lat_ref, keys_ref, vals_ref,
              wse, bse, wc1, bc1, wc2, bc2, wq, bq,
              wagg, bagg, wk1, bk1, wk2, bk2, wv1, bv1, wv2, bv2,
              mk_ref, mv_ref, sim_s):
    f32 = jnp.float32
    parity = pl.program_id(0) % 2

    def dot(a, b):
        return jax.lax.dot_general(a, b, (((1,), (0,)), ((), ())),
                                   preferred_element_type=f32)

    s = dot(state_ref[...], wse[...]) + bse[...]
    qc = jnp.concatenate([s, lat_ref[...]], axis=1)
    qc = dot(qc, wc1[...]) + bc1[...]
    qc = dot(qc, wc2[...]) + bc2[...]
    q_heads = [dot(qc, wq[:, _K * h:_K * (h + 1)]) + bq[:, _K * h:_K * (h + 1)]
               for h in range(_H)]

    qstack = jnp.concatenate(q_heads, axis=0)
    row_b = jax.lax.broadcasted_iota(jnp.int32, (_ROWS, 1), 0) % _BB
    qbd = jnp.concatenate(
        [jnp.where(row_b == i, qstack, 0.0) for i in range(_BB)],
        axis=1)
    keys_m = keys_ref[...].reshape(_E, _BB * _K)
    sim_a = jax.lax.dot_general(qbd, keys_m, (((1,), (1,)), ((), ())),
                                preferred_element_type=f32)
    sim_s[parity] = sim_a * (1.0 / 16.0)

    sim = sim_s[1 - parity]

    ub = jax.lax.bitcast_convert_type(sim, jnp.uint32)
    u = jnp.where(ub >= jnp.uint32(0x80000000), ~ub,
                  ub | jnp.uint32(0x80000000))
    def cntf(ge):
        return jnp.sum(jnp.where(ge, 1.0, 0.0), axis=1, keepdims=True)

    kth = float(_TOPK) - 0.5
    prefix = jnp.zeros((_ROWS, 1), jnp.uint32)
    for st in range(16):
        sh = 30 - 2 * st
        chi = prefix | jnp.uint32(2 << sh)
        clo = prefix | jnp.uint32(1 << sh)
        cboth = prefix | jnp.uint32(3 << sh)
        nhi = cntf(u >= chi)
        nlo = cntf(u >= clo)
        nboth = cntf(u >= cboth)
        prefix = jnp.where(nhi >= kth,
                           jnp.where(nboth >= kth, cboth, chi),
                           jnp.where(nlo >= kth, clo, prefix))
    thr = prefix

    gt = u > thr
    need = float(_TOPK) - cntf(gt)
    eq = u == thr
    ri = jax.lax.broadcasted_iota(jnp.int32, (_E, _E), 0)
    ci = jax.lax.broadcasted_iota(jnp.int32, (_E, _E), 1)
    ltri = (ri < ci).astype(f32)
    excl = dot(eq.astype(f32), ltri)
    mask = gt | (eq & (excl < need))

    m = jnp.max(sim, axis=1, keepdims=True)
    p = jnp.where(mask, jnp.exp(sim - m), 0.0)
    w = p / jnp.sum(p, axis=1, keepdims=True)

    r_bd = dot(w, vals_ref[...].reshape(_E, _BB * _V))
    r_all = jnp.zeros((_ROWS, _V), f32)
    for i in range(_BB):
        r_all = jnp.where(row_b == i, r_bd[:, _V * i:_V * (i + 1)], r_all)

    acc = bagg[...]
    for h in range(_H):
        acc = acc + dot(r_all[_BB * h:_BB * (h + 1), :],
                        wagg[_V * h:_V * (h + 1), :])

    mk = dot(dot(acc, wk1[...]) + bk1[...], wk2[...]) + bk2[...]
    mv = dot(dot(acc, wv1[...]) + bv1[...], wv2[...]) + bv2[...]
    mk_ref[...] = mk
    mv_ref[...] = mv


def kernel(state, task_inference_latent, mem_keys, mem_vals, params):
    f32 = jnp.float32
    wse = params["state_encoder"][0]["w"]
    c1, c2 = params["concat_query_encoder"]
    qe = params["query_encoder"][0]
    agg = params["value_aggregator"][0]
    k1, k2 = params["read_memory_to_key"]
    v1, v2 = params["read_memory_to_value"]

    def bb(b):
        return jnp.broadcast_to(b.astype(f32), (_BB, b.shape[0]))

    nblk = _B // _BB
    grid = nblk + 1
    cur = lambda i: jnp.minimum(i, nblk - 1)
    prev = lambda i: jnp.maximum(i - 1, 0)
    full = lambda a: pl.BlockSpec(a.shape, lambda i: (0,) * a.ndim)
    in_specs = [
        pl.BlockSpec((_BB, state.shape[1]), lambda i: (cur(i), 0)),
        pl.BlockSpec((_BB, task_inference_latent.shape[1]),
                     lambda i: (cur(i), 0)),
        pl.BlockSpec((_E, _BB, _K), lambda i: (0, cur(i), 0)),
        pl.BlockSpec((_E, _BB, _V), lambda i: (0, prev(i), 0)),
    ]
    weights = []
    for wmat, bvec in ((wse, params["state_encoder"][0]["b"]),
                       (c1["w"], c1["b"]), (c2["w"], c2["b"]),
                       (qe["w"], qe["b"]), (agg["w"], agg["b"]),
                       (k1["w"], k1["b"]), (k2["w"], k2["b"]),
                       (v1["w"], v1["b"]), (v2["w"], v2["b"])):
        weights.append(wmat.astype(f32))
        weights.append(bb(bvec))
    in_specs += [full(a) for a in weights]

    out_shape = (jax.ShapeDtypeStruct((_B, k2["w"].shape[1]), f32),
                 jax.ShapeDtypeStruct((_B, v2["w"].shape[1]), f32))
    out_specs = (pl.BlockSpec((_BB, k2["w"].shape[1]), lambda i: (prev(i), 0)),
                 pl.BlockSpec((_BB, v2["w"].shape[1]), lambda i: (prev(i), 0)))

    mk, mv = pl.pallas_call(
        _dnd_body,
        grid=(grid,),
        in_specs=in_specs,
        out_specs=out_specs,
        out_shape=out_shape,
        scratch_shapes=[pltpu.VMEM((2, _ROWS, _E), jnp.float32)],
    )(state, task_inference_latent, mem_keys, mem_vals, *weights)
    return mk, mv


# speedup vs baseline: 1.3799x; 1.3799x over previous
import jax
import jax.numpy as jnp
from jax.experimental import pallas as pl
from jax.experimental.pallas import tpu as pltpu

_B = 1024
_BB = 8
_H = 8
_K = 256
_E = 200
_V = 256
_TOPK = 32
_ROWS = _BB * _H


def _dnd_body(state_ref, lat_ref, keys_ref, vals_ref,
              wse, bse, wc1, bc1, wc2, bc2, wq, bq,
              wagg, bagg, wk1, bk1, wk2, bk2, wv1, bv1, wv2, bv2,
              mk_ref, mv_ref, sim_s):
    f32 = jnp.float32
    sim = sim_s[...]

    def dot(a, b):
        return jax.lax.dot_general(a, b, (((1,), (0,)), ((), ())),
                                   preferred_element_type=f32)

    s = dot(state_ref[...], wse[...]) + bse[...]
    qc = jnp.concatenate([s, lat_ref[...]], axis=1)
    qc = dot(qc, wc1[...]) + bc1[...]
    qc = dot(qc, wc2[...]) + bc2[...]
    q_heads = [dot(qc, wq[:, _K * h:_K * (h + 1)]) + bq[:, _K * h:_K * (h + 1)]
               for h in range(_H)]

    qstack = jnp.concatenate(q_heads, axis=0)
    row_b = jax.lax.broadcasted_iota(jnp.int32, (_ROWS, 1), 0) % _BB
    qbd = jnp.concatenate(
        [jnp.where(row_b == i, qstack, 0.0) for i in range(_BB)],
        axis=1)
    keys_m = keys_ref[...].reshape(_E, _BB * _K)
    sim_a = jax.lax.dot_general(qbd, keys_m, (((1,), (1,)), ((), ())),
                                preferred_element_type=f32)
    sim_s[...] = sim_a * (1.0 / 16.0)

    ub = jax.lax.bitcast_convert_type(sim, jnp.uint32)
    u = jnp.where(ub >= jnp.uint32(0x80000000), ~ub,
                  ub | jnp.uint32(0x80000000))
    def cntf(ge):
        return jnp.sum(jnp.where(ge, 1.0, 0.0), axis=1, keepdims=True)

    kth = float(_TOPK) - 0.5
    prefix = jnp.zeros((_ROWS, 1), jnp.uint32)
    for st in range(16):
        sh = 30 - 2 * st
        chi = prefix | jnp.uint32(2 << sh)
        clo = prefix | jnp.uint32(1 << sh)
        cboth = prefix | jnp.uint32(3 << sh)
        nhi = cntf(u >= chi)
        nlo = cntf(u >= clo)
        nboth = cntf(u >= cboth)
        prefix = jnp.where(nhi >= kth,
                           jnp.where(nboth >= kth, cboth, chi),
                           jnp.where(nlo >= kth, clo, prefix))
    thr = prefix

    gt = u > thr
    need = float(_TOPK) - cntf(gt)
    eq = u == thr
    ri = jax.lax.broadcasted_iota(jnp.int32, (_E, _E), 0)
    ci = jax.lax.broadcasted_iota(jnp.int32, (_E, _E), 1)
    ltri = (ri < ci).astype(f32)
    excl = dot(eq.astype(f32), ltri)
    mask = gt | (eq & (excl < need))

    m = jnp.max(sim, axis=1, keepdims=True)
    p = jnp.where(mask, jnp.exp(sim - m), 0.0)
    w = p / jnp.sum(p, axis=1, keepdims=True)

    r_bd = dot(w, vals_ref[...].reshape(_E, _BB * _V))
    r_all = jnp.zeros((_ROWS, _V), f32)
    for i in range(_BB):
        r_all = jnp.where(row_b == i, r_bd[:, _V * i:_V * (i + 1)], r_all)

    acc = bagg[...]
    for h in range(_H):
        acc = acc + dot(r_all[_BB * h:_BB * (h + 1), :],
                        wagg[_V * h:_V * (h + 1), :])

    mk = dot(dot(acc, wk1[...]) + bk1[...], wk2[...]) + bk2[...]
    mv = dot(dot(acc, wv1[...]) + bv1[...], wv2[...]) + bv2[...]
    mk_ref[...] = mk
    mv_ref[...] = mv


def kernel(state, task_inference_latent, mem_keys, mem_vals, params):
    f32 = jnp.float32
    wse = params["state_encoder"][0]["w"]
    c1, c2 = params["concat_query_encoder"]
    qe = params["query_encoder"][0]
    agg = params["value_aggregator"][0]
    k1, k2 = params["read_memory_to_key"]
    v1, v2 = params["read_memory_to_value"]

    def bb(b):
        return jnp.broadcast_to(b.astype(f32), (_BB, b.shape[0]))

    nblk = _B // _BB
    grid = nblk + 1
    cur = lambda i: jnp.minimum(i, nblk - 1)
    prev = lambda i: jnp.maximum(i - 1, 0)
    full = lambda a: pl.BlockSpec(a.shape, lambda i: (0,) * a.ndim)
    in_specs = [
        pl.BlockSpec((_BB, state.shape[1]), lambda i: (cur(i), 0)),
        pl.BlockSpec((_BB, task_inference_latent.shape[1]),
                     lambda i: (cur(i), 0)),
        pl.BlockSpec((_E, _BB, _K), lambda i: (0, cur(i), 0)),
        pl.BlockSpec((_E, _BB, _V), lambda i: (0, prev(i), 0)),
    ]
    weights = []
    for wmat, bvec in ((wse, params["state_encoder"][0]["b"]),
                       (c1["w"], c1["b"]), (c2["w"], c2["b"]),
                       (qe["w"], qe["b"]), (agg["w"], agg["b"]),
                       (k1["w"], k1["b"]), (k2["w"], k2["b"]),
                       (v1["w"], v1["b"]), (v2["w"], v2["b"])):
        weights.append(wmat.astype(f32))
        weights.append(bb(bvec))
    in_specs += [full(a) for a in weights]

    out_shape = (jax.ShapeDtypeStruct((_B, k2["w"].shape[1]), f32),
                 jax.ShapeDtypeStruct((_B, v2["w"].shape[1]), f32))
    out_specs = (pl.BlockSpec((_BB, k2["w"].shape[1]), lambda i: (prev(i), 0)),
                 pl.BlockSpec((_BB, v2["w"].shape[1]), lambda i: (prev(i), 0)))

    mk, mv = pl.pallas_call(
        _dnd_body,
        grid=(grid,),
        in_specs=in_specs,
        out_specs=out_specs,
        out_shape=out_shape,
        scratch_shapes=[pltpu.VMEM((_ROWS, _E), jnp.float32)],
    )(state, task_inference_latent, mem_keys, mem_vals, *weights)
    return mk, mv


# speedup vs baseline: 1.4393x; 1.0431x over previous
import jax
import jax.numpy as jnp
from jax.experimental import pallas as pl
from jax.experimental.pallas import tpu as pltpu

_B = 1024
_BB = 8
_H = 8
_K = 256
_E = 200
_V = 256
_TOPK = 32
_ROWS = _BB * _H


def _dnd_body(state_ref, lat_ref, keys_ref, vals_ref,
              wse, bse, wc1, bc1, wc2, bc2, wq, bq,
              wagg, bagg, wk1, bk1, wk2, bk2, wv1, bv1, wv2, bv2,
              mk_ref, mv_ref, sim_s):
    f32 = jnp.float32
    sim = sim_s[...]

    def dot(a, b):
        return jax.lax.dot_general(a, b, (((1,), (0,)), ((), ())),
                                   preferred_element_type=f32)

    s = dot(state_ref[...], wse[...]) + bse[...]
    qc = jnp.concatenate([s, lat_ref[...]], axis=1)
    qc = dot(qc, wc1[...]) + bc1[...]
    qc = dot(qc, wc2[...]) + bc2[...]
    q_heads = [dot(qc, wq[:, _K * h:_K * (h + 1)]) + bq[:, _K * h:_K * (h + 1)]
               for h in range(_H)]

    qstack = jnp.concatenate(q_heads, axis=0)
    row_b = jax.lax.broadcasted_iota(jnp.int32, (_ROWS, 1), 0) % _BB
    qbd = jnp.concatenate(
        [jnp.where(row_b == i, qstack, 0.0) for i in range(_BB)],
        axis=1)
    keys_m = keys_ref[...].reshape(_E, _BB * _K)
    sim_a = jax.lax.dot_general(qbd, keys_m, (((1,), (1,)), ((), ())),
                                preferred_element_type=f32)
    sim_s[...] = sim_a * (1.0 / 16.0)

    ub = jax.lax.bitcast_convert_type(sim, jnp.uint32)
    u = jnp.where(ub >= jnp.uint32(0x80000000), ~ub,
                  ub | jnp.uint32(0x80000000))
    def cntf(ge):
        return jnp.sum(jnp.where(ge, 1.0, 0.0), axis=1, keepdims=True)

    kth = float(_TOPK) - 0.5
    prefix = jnp.zeros((_ROWS, 1), jnp.uint32)
    for sh, nbits in [(30, 2)] + [(27 - 3 * s, 3) for s in range(10)]:
        top = (1 << nbits) - 1
        cands = [prefix | jnp.uint32(m << sh) for m in range(1, top + 1)]
        ge = [cntf(u >= c) >= kth for c in cands]

        def pick(lo_m, hi_m):
            if lo_m == hi_m:
                return cands[lo_m - 1] if lo_m >= 1 else prefix
            mid = (lo_m + hi_m + 1) // 2
            return jnp.where(ge[mid - 1], pick(mid, hi_m), pick(lo_m, mid - 1))

        prefix = pick(0, top)
    thr = prefix

    gt = u > thr
    need = float(_TOPK) - cntf(gt)
    eq = u == thr
    ri = jax.lax.broadcasted_iota(jnp.int32, (_E, _E), 0)
    ci = jax.lax.broadcasted_iota(jnp.int32, (_E, _E), 1)
    ltri = (ri < ci).astype(f32)
    excl = dot(eq.astype(f32), ltri)
    mask = gt | (eq & (excl < need))

    m = jnp.max(sim, axis=1, keepdims=True)
    p = jnp.where(mask, jnp.exp(sim - m), 0.0)
    w = p / jnp.sum(p, axis=1, keepdims=True)

    r_bd = dot(w, vals_ref[...].reshape(_E, _BB * _V))
    r_all = jnp.zeros((_ROWS, _V), f32)
    for i in range(_BB):
        r_all = jnp.where(row_b == i, r_bd[:, _V * i:_V * (i + 1)], r_all)

    acc = bagg[...]
    for h in range(_H):
        acc = acc + dot(r_all[_BB * h:_BB * (h + 1), :],
                        wagg[_V * h:_V * (h + 1), :])

    mk = dot(dot(acc, wk1[...]) + bk1[...], wk2[...]) + bk2[...]
    mv = dot(dot(acc, wv1[...]) + bv1[...], wv2[...]) + bv2[...]
    mk_ref[...] = mk
    mv_ref[...] = mv


def kernel(state, task_inference_latent, mem_keys, mem_vals, params):
    f32 = jnp.float32
    wse = params["state_encoder"][0]["w"]
    c1, c2 = params["concat_query_encoder"]
    qe = params["query_encoder"][0]
    agg = params["value_aggregator"][0]
    k1, k2 = params["read_memory_to_key"]
    v1, v2 = params["read_memory_to_value"]

    def bb(b):
        return jnp.broadcast_to(b.astype(f32), (_BB, b.shape[0]))

    nblk = _B // _BB
    grid = nblk + 1
    cur = lambda i: jnp.minimum(i, nblk - 1)
    prev = lambda i: jnp.maximum(i - 1, 0)
    full = lambda a: pl.BlockSpec(a.shape, lambda i: (0,) * a.ndim)
    in_specs = [
        pl.BlockSpec((_BB, state.shape[1]), lambda i: (cur(i), 0)),
        pl.BlockSpec((_BB, task_inference_latent.shape[1]),
                     lambda i: (cur(i), 0)),
        pl.BlockSpec((_E, _BB, _K), lambda i: (0, cur(i), 0)),
        pl.BlockSpec((_E, _BB, _V), lambda i: (0, prev(i), 0)),
    ]
    weights = []
    for wmat, bvec in ((wse, params["state_encoder"][0]["b"]),
                       (c1["w"], c1["b"]), (c2["w"], c2["b"]),
                       (qe["w"], qe["b"]), (agg["w"], agg["b"]),
                       (k1["w"], k1["b"]), (k2["w"], k2["b"]),
                       (v1["w"], v1["b"]), (v2["w"], v2["b"])):
        weights.append(wmat.astype(f32))
        weights.append(bb(bvec))
    in_specs += [full(a) for a in weights]

    out_shape = (jax.ShapeDtypeStruct((_B, k2["w"].shape[1]), f32),
                 jax.ShapeDtypeStruct((_B, v2["w"].shape[1]), f32))
    out_specs = (pl.BlockSpec((_BB, k2["w"].shape[1]), lambda i: (prev(i), 0)),
                 pl.BlockSpec((_BB, v2["w"].shape[1]), lambda i: (prev(i), 0)))

    mk, mv = pl.pallas_call(
        _dnd_body,
        grid=(grid,),
        in_specs=in_specs,
        out_specs=out_specs,
        out_shape=out_shape,
        scratch_shapes=[pltpu.VMEM((_ROWS, _E), jnp.float32)],
    )(state, task_inference_latent, mem_keys, mem_vals, *weights)
    return mk, mv
